# Initial kernel scaffold; baseline (speedup 1.0000x reference)
import jax
import jax.numpy as jnp
from jax.experimental import pallas as pl


def kernel(x, edge_index, edge_vec, pre_W, pre_b, pre_a, gc_mlp2_W, gc_mlp2_b, gc_mlp2_a, gc_mlp_W, gc_mlp_b, gc_mlp_a, bn_gamma, bn_beta, dos_W1, dos_b1, dos_a1, dos_W2, dos_b2, dos_a2, sc_W1, sc_b1, sc_a1, sc_W2, sc_b2):
    raise NotImplementedError("write your pallas kernel here")


# speedup vs baseline: 3.5063x; 3.5063x over previous
import functools

import jax
import jax.numpy as jnp
from jax import lax
from jax.experimental import pallas as pl
from jax.experimental.pallas import tpu as pltpu
from jax.experimental.pallas import tpu_sc as plsc

N = 10000
E = 320000
D = 128
DE = 50
OUT = 200
GC = 3

NC = 2
NS = 16
NW = NC * NS
L = 16
CW = 128

EPW = E // NW
K = 80
NCHUNK = EPW // K
NP = 10240
RPS = NP // NS

F32 = jnp.float32


def _sc_edge_body(a_hbm, b_hbm, c_hbm, src_hbm, dst_hbm, alpha_hbm,
                  zero_nd_hbm,
                  ssum_out,
                  idx_s, idx_d, arows, brows, crows, alpha_v,
                  acc, sem):
    cid = lax.axis_index("c")
    sid = lax.axis_index("s")
    wid = sid * NC + cid

    r0 = sid * RPS
    pltpu.sync_copy(zero_nd_hbm.at[pl.ds(r0, RPS)], acc.at[pl.ds(r0, RPS)])
    pltpu.sync_copy(alpha_hbm, alpha_v)
    plsc.subcore_barrier()

    base = wid * EPW

    def chunk(ci, carry):
        off = base + ci * K
        pltpu.sync_copy(dst_hbm.at[pl.ds(off, K)], idx_d)
        pltpu.sync_copy(src_hbm.at[pl.ds(off, K)], idx_s)
        cp_a = pltpu.async_copy(a_hbm.at[idx_d], arows, sem)
        cp_b = pltpu.async_copy(b_hbm.at[idx_s], brows, sem)
        cp_c = pltpu.async_copy(c_hbm.at[pl.ds(off, K)], crows, sem)
        cp_a.wait()
        cp_b.wait()
        cp_c.wait()
        al = alpha_v[...]

        def row(r, c2):
            for g in range(D // L):
                s = pl.ds(g * L, L)
                u = arows[r, s] + brows[r, s] + crows[r, s]
                arows[r, s] = jnp.where(u >= 0.0, u, al * u)
            return c2

        lax.fori_loop(0, K, row, 0)
        pltpu.sync_copy(arows, acc.at[idx_d], add=True)
        return carry

    lax.fori_loop(0, NCHUNK, chunk, 0)
    plsc.subcore_barrier()
    pltpu.sync_copy(acc.at[pl.ds(r0, RPS)], ssum_out.at[cid, pl.ds(r0, RPS)])


_sc_edge = pl.kernel(
    _sc_edge_body,
    out_type=jax.ShapeDtypeStruct((NC, NP, D), F32),
    mesh=plsc.VectorSubcoreMesh(core_axis_name="c", subcore_axis_name="s"),
    scratch_types=[
        pltpu.VMEM((K,), jnp.int32),
        pltpu.VMEM((K,), jnp.int32),
        pltpu.VMEM((K, D), F32),
        pltpu.VMEM((K, D), F32),
        pltpu.VMEM((K, D), F32),
        pltpu.VMEM((L,), F32),
        pltpu.VMEM_SHARED((NP, D), F32),
        pltpu.SemaphoreType.DMA,
    ],
)


def _sc_cnt_body(dst_hbm, zero_cnt_hbm, ones_hbm, cnt_out,
                 idx_d, ones_v, cacc):
    cid = lax.axis_index("c")
    sid = lax.axis_index("s")
    wid = sid * NC + cid
    r0 = sid * RPS
    pltpu.sync_copy(zero_cnt_hbm.at[pl.ds(r0, RPS)], cacc.at[pl.ds(r0, RPS)])
    pltpu.sync_copy(ones_hbm, ones_v)
    plsc.subcore_barrier()
    base = wid * EPW

    def chunk(ci, carry):
        pltpu.sync_copy(dst_hbm.at[pl.ds(base + ci * K, K)], idx_d)
        pltpu.sync_copy(ones_v, cacc.at[idx_d], add=True)
        return carry

    lax.fori_loop(0, NCHUNK, chunk, 0)
    plsc.subcore_barrier()
    pltpu.sync_copy(cacc.at[pl.ds(r0, RPS)], cnt_out.at[cid, pl.ds(r0, RPS)])


_sc_cnt = pl.kernel(
    _sc_cnt_body,
    out_type=jax.ShapeDtypeStruct((NC, NP, CW), F32),
    mesh=plsc.VectorSubcoreMesh(core_axis_name="c", subcore_axis_name="s"),
    scratch_types=[
        pltpu.VMEM((K,), jnp.int32),
        pltpu.VMEM((K, CW), F32),
        pltpu.VMEM_SHARED((NP, CW), F32),
    ],
)


def _prep_body(x_ref, w_ref, b_ref, a_ref, wd_ref, ws_ref,
               out_ref, ao_ref, bo_ref):
    h = jnp.dot(x_ref[...], w_ref[...], preferred_element_type=F32) + b_ref[...]
    o = jnp.where(h >= 0.0, h, a_ref[0, 0] * h)
    out_ref[...] = o
    ao_ref[...] = jnp.dot(o, wd_ref[...], preferred_element_type=F32)
    bo_ref[...] = jnp.dot(o, ws_ref[...], preferred_element_type=F32)


def _edge_feat_body(ev_ref, w2_ref, b2_ref, a2_ref, we_ref, bm_ref, c_ref):
    e = jnp.dot(ev_ref[...], w2_ref[...], preferred_element_type=F32) + b2_ref[...]
    e = jnp.where(e >= 0.0, e, a2_ref[0, 0] * e)
    c_ref[...] = jnp.dot(e, we_ref[...], preferred_element_type=F32) + bm_ref[...]


def _cnt_inv_body(cnt_ref, inv_ref):
    c = cnt_ref[0, :N, 0:1] + cnt_ref[1, :N, 0:1]
    inv_ref[...] = 1.0 / jnp.maximum(c, 1.0)


def _node_body_ab(ssum_ref, inv_ref, prev_ref, gam_ref, bet_ref,
                  wd_ref, ws_ref, out_ref, ao_ref, bo_ref):
    s = ssum_ref[0, :N] + ssum_ref[1, :N]
    o = s * inv_ref[...] + prev_ref[...]
    mean = jnp.mean(o, axis=0, keepdims=True)
    var = jnp.mean((o - mean) * (o - mean), axis=0, keepdims=True)
    o = (o - mean) / jnp.sqrt(var + 1e-5) * gam_ref[...] + bet_ref[...]
    out_ref[...] = o
    ao_ref[...] = jnp.dot(o, wd_ref[...], preferred_element_type=F32)
    bo_ref[...] = jnp.dot(o, ws_ref[...], preferred_element_type=F32)


def _node_body_last(ssum_ref, inv_ref, prev_ref, gam_ref, bet_ref, out_ref):
    s = ssum_ref[0, :N] + ssum_ref[1, :N]
    o = s * inv_ref[...] + prev_ref[...]
    mean = jnp.mean(o, axis=0, keepdims=True)
    var = jnp.mean((o - mean) * (o - mean), axis=0, keepdims=True)
    out_ref[...] = (o - mean) / jnp.sqrt(var + 1e-5) * gam_ref[...] + bet_ref[...]


def _heads_body(o_ref, dw1_ref, db1_ref, da1_ref, dw2_ref, db2_ref, da2_ref,
                sw1_ref, sb1_ref, sa1_ref, sw2_ref, sb2_ref,
                dos_ref, scal_ref):
    o = o_ref[...]
    h = jnp.dot(o, dw1_ref[...], preferred_element_type=F32) + db1_ref[...]
    h = jnp.where(h >= 0.0, h, da1_ref[0, 0] * h)
    d = jnp.dot(h, dw2_ref[...], preferred_element_type=F32) + db2_ref[...]
    dos_ref[...] = jnp.where(d >= 0.0, d, da2_ref[0, 0] * d)
    s = jnp.dot(o, sw1_ref[...], preferred_element_type=F32) + sb1_ref[...]
    s = jnp.where(s >= 0.0, s, sa1_ref[0, 0] * s)
    scal_ref[...] = jnp.dot(s, sw2_ref[...], preferred_element_type=F32) + sb2_ref[...]


_prep = pl.pallas_call(
    _prep_body,
    out_shape=[
        jax.ShapeDtypeStruct((N, D), F32),
        jax.ShapeDtypeStruct((N, D), F32),
        jax.ShapeDtypeStruct((N, D), F32),
    ],
)

_BE = 4000

_edge_feat = pl.pallas_call(
    _edge_feat_body,
    grid=(E // _BE,),
    in_specs=[
        pl.BlockSpec((_BE, DE), lambda i: (i, 0)),
        pl.BlockSpec((DE, DE), lambda i: (0, 0)),
        pl.BlockSpec((1, DE), lambda i: (0, 0)),
        pl.BlockSpec((1, 1), lambda i: (0, 0)),
        pl.BlockSpec((DE, D), lambda i: (0, 0)),
        pl.BlockSpec((1, D), lambda i: (0, 0)),
    ],
    out_specs=pl.BlockSpec((_BE, D), lambda i: (i, 0)),
    out_shape=jax.ShapeDtypeStruct((E, D), F32),
)

_cnt_inv = pl.pallas_call(
    _cnt_inv_body,
    out_shape=jax.ShapeDtypeStruct((N, 1), F32),
)

_node_ab = pl.pallas_call(
    _node_body_ab,
    out_shape=[
        jax.ShapeDtypeStruct((N, D), F32),
        jax.ShapeDtypeStruct((N, D), F32),
        jax.ShapeDtypeStruct((N, D), F32),
    ],
)

_node_last = pl.pallas_call(
    _node_body_last,
    out_shape=jax.ShapeDtypeStruct((N, D), F32),
)

_heads = pl.pallas_call(
    _heads_body,
    out_shape=[
        jax.ShapeDtypeStruct((N, OUT), F32),
        jax.ShapeDtypeStruct((N, 1), F32),
    ],
)


def kernel(x, edge_index, edge_vec, pre_W, pre_b, pre_a,
           gc_mlp2_W, gc_mlp2_b, gc_mlp2_a, gc_mlp_W, gc_mlp_b, gc_mlp_a,
           bn_gamma, bn_beta, dos_W1, dos_b1, dos_a1, dos_W2, dos_b2, dos_a2,
           sc_W1, sc_b1, sc_a1, sc_W2, sc_b2):
    src = edge_index[0]
    dst = edge_index[1]
    wd = gc_mlp_W[:, :D, :]
    ws = gc_mlp_W[:, D:2 * D, :]
    we = gc_mlp_W[:, 2 * D:, :]

    zero_nd = jnp.zeros((NP, D), F32)
    zero_cnt = jnp.zeros((NP, CW), F32)
    ones_rows = jnp.ones((K, CW), F32)

    out, a_rows, b_rows = _prep(
        x, pre_W, pre_b.reshape(1, D), pre_a.reshape(1, 1), wd[0], ws[0])
    inv = _cnt_inv(_sc_cnt(dst, zero_cnt, ones_rows))

    for i in range(GC):
        c_rows = _edge_feat(
            edge_vec, gc_mlp2_W[i], gc_mlp2_b[i].reshape(1, DE),
            gc_mlp2_a[i].reshape(1, 1), we[i], gc_mlp_b[i].reshape(1, D))
        alpha = jnp.broadcast_to(gc_mlp_a[i], (L,)).astype(F32)
        ssum2 = _sc_edge(a_rows, b_rows, c_rows, src, dst, alpha, zero_nd)
        gam = bn_gamma[i].reshape(1, D)
        bet = bn_beta[i].reshape(1, D)
        if i < GC - 1:
            out, a_rows, b_rows = _node_ab(
                ssum2, inv, out, gam, bet, wd[i + 1], ws[i + 1])
        else:
            out = _node_last(ssum2, inv, out, gam, bet)

    dos, scal = _heads(
        out, dos_W1, dos_b1.reshape(1, 64), dos_a1.reshape(1, 1),
        dos_W2, dos_b2.reshape(1, OUT), dos_a2.reshape(1, 1),
        sc_W1, sc_b1.reshape(1, 100), sc_a1.reshape(1, 1),
        sc_W2, sc_b2.reshape(1, 1))
    return dos, scal.reshape(N)


# speedup vs baseline: 4.0891x; 1.1662x over previous
import functools

import jax
import jax.numpy as jnp
from jax import lax
from jax.experimental import pallas as pl
from jax.experimental.pallas import tpu as pltpu
from jax.experimental.pallas import tpu_sc as plsc

N = 10000
E = 320000
D = 128
DE = 50
OUT = 200
GC = 3

NC = 2
NS = 16
NW = NC * NS
L = 16
CW = 128

EPW = E // NW
K = 80
NCHUNK = EPW // K
SG = 2000
NP = 10240
RPS = NP // NS

F32 = jnp.float32


def _sc_edge_body(a_hbm, b_hbm, c_hbm, src_hbm, dst_hbm, alpha_hbm,
                  zero_nd_hbm,
                  ssum_out,
                  idx_sa, idx_da, idx_d, arows, brows, crows, alpha_v,
                  acc, sem):
    cid = lax.axis_index("c")
    sid = lax.axis_index("s")
    wid = sid * NC + cid

    r0 = sid * RPS
    pltpu.sync_copy(zero_nd_hbm.at[pl.ds(r0, RPS)], acc.at[pl.ds(r0, RPS)])
    pltpu.sync_copy(alpha_hbm, alpha_v)
    plsc.subcore_barrier()

    base = wid * EPW

    def stage(st, carry0):
        soff = base + st * SG
        pltpu.sync_copy(dst_hbm.at[pl.ds(soff, SG)], idx_da)
        pltpu.sync_copy(src_hbm.at[pl.ds(soff, SG)], idx_sa)

        def chunk(ci, carry):
            for g in range(K // L):
                idx_d[pl.ds(g * L, L)] = idx_da[pl.ds(ci * K + g * L, L)]
            cp_a = pltpu.async_copy(a_hbm.at[idx_d], arows, sem)
            cp_b = pltpu.async_copy(
                b_hbm.at[idx_sa.at[pl.ds(ci * K, K)]], brows, sem)
            cp_c = pltpu.async_copy(
                c_hbm.at[pl.ds(soff + ci * K, K)], crows, sem)
            cp_a.wait()
            cp_b.wait()
            cp_c.wait()
            al = alpha_v[...]

            def row(r, c2):
                for g in range(D // L):
                    s = pl.ds(g * L, L)
                    u = arows[r, s] + brows[r, s] + crows[r, s]
                    arows[r, s] = jnp.where(u >= 0.0, u, al * u)
                return c2

            lax.fori_loop(0, K, row, 0)
            pltpu.sync_copy(arows, acc.at[idx_d], add=True)
            return carry

        lax.fori_loop(0, SG // K, chunk, 0)
        return carry0

    lax.fori_loop(0, EPW // SG, stage, 0)
    plsc.subcore_barrier()
    pltpu.sync_copy(acc.at[pl.ds(r0, RPS)], ssum_out.at[cid, pl.ds(r0, RPS)])


_sc_edge = pl.kernel(
    _sc_edge_body,
    out_type=jax.ShapeDtypeStruct((NC, NP, D), F32),
    mesh=plsc.VectorSubcoreMesh(core_axis_name="c", subcore_axis_name="s"),
    scratch_types=[
        pltpu.VMEM((SG,), jnp.int32),
        pltpu.VMEM((SG,), jnp.int32),
        pltpu.VMEM((K,), jnp.int32),
        pltpu.VMEM((K, D), F32),
        pltpu.VMEM((K, D), F32),
        pltpu.VMEM((K, D), F32),
        pltpu.VMEM((L,), F32),
        pltpu.VMEM_SHARED((NP, D), F32),
        pltpu.SemaphoreType.DMA,
    ],
)


def _sc_cnt_body(dst_hbm, zero_cnt_hbm, ones_hbm, cnt_out,
                 idx_d, ones_v, cacc):
    cid = lax.axis_index("c")
    sid = lax.axis_index("s")
    wid = sid * NC + cid
    r0 = sid * RPS
    pltpu.sync_copy(zero_cnt_hbm.at[pl.ds(r0, RPS)], cacc.at[pl.ds(r0, RPS)])
    pltpu.sync_copy(ones_hbm, ones_v)
    plsc.subcore_barrier()
    base = wid * EPW

    def chunk(ci, carry):
        pltpu.sync_copy(dst_hbm.at[pl.ds(base + ci * K, K)], idx_d)
        pltpu.sync_copy(ones_v, cacc.at[idx_d], add=True)
        return carry

    lax.fori_loop(0, NCHUNK, chunk, 0)
    plsc.subcore_barrier()
    pltpu.sync_copy(cacc.at[pl.ds(r0, RPS)], cnt_out.at[cid, pl.ds(r0, RPS)])


_sc_cnt = pl.kernel(
    _sc_cnt_body,
    out_type=jax.ShapeDtypeStruct((NC, NP, CW), F32),
    mesh=plsc.VectorSubcoreMesh(core_axis_name="c", subcore_axis_name="s"),
    scratch_types=[
        pltpu.VMEM((K,), jnp.int32),
        pltpu.VMEM((K, CW), F32),
        pltpu.VMEM_SHARED((NP, CW), F32),
    ],
)


def _prep_body(x_ref, w_ref, b_ref, a_ref, wd_ref, ws_ref,
               out_ref, ao_ref, bo_ref):
    h = jnp.dot(x_ref[...], w_ref[...], preferred_element_type=F32) + b_ref[...]
    o = jnp.where(h >= 0.0, h, a_ref[0, 0] * h)
    out_ref[...] = o
    ao_ref[...] = jnp.dot(o, wd_ref[...], preferred_element_type=F32)
    bo_ref[...] = jnp.dot(o, ws_ref[...], preferred_element_type=F32)


def _edge_feat_body(ev_ref, w2_ref, b2_ref, a2_ref, we_ref, bm_ref, c_ref):
    e = jnp.dot(ev_ref[...], w2_ref[...], preferred_element_type=F32) + b2_ref[...]
    e = jnp.where(e >= 0.0, e, a2_ref[0, 0] * e)
    c_ref[...] = jnp.dot(e, we_ref[...], preferred_element_type=F32) + bm_ref[...]


def _cnt_inv_body(cnt_ref, inv_ref):
    c = cnt_ref[0, :N, 0:1] + cnt_ref[1, :N, 0:1]
    inv_ref[...] = 1.0 / jnp.maximum(c, 1.0)


def _node_body_ab(ssum_ref, inv_ref, prev_ref, gam_ref, bet_ref,
                  wd_ref, ws_ref, out_ref, ao_ref, bo_ref):
    s = ssum_ref[0, :N] + ssum_ref[1, :N]
    o = s * inv_ref[...] + prev_ref[...]
    mean = jnp.mean(o, axis=0, keepdims=True)
    var = jnp.mean((o - mean) * (o - mean), axis=0, keepdims=True)
    o = (o - mean) / jnp.sqrt(var + 1e-5) * gam_ref[...] + bet_ref[...]
    out_ref[...] = o
    ao_ref[...] = jnp.dot(o, wd_ref[...], preferred_element_type=F32)
    bo_ref[...] = jnp.dot(o, ws_ref[...], preferred_element_type=F32)


def _node_body_last(ssum_ref, inv_ref, prev_ref, gam_ref, bet_ref, out_ref):
    s = ssum_ref[0, :N] + ssum_ref[1, :N]
    o = s * inv_ref[...] + prev_ref[...]
    mean = jnp.mean(o, axis=0, keepdims=True)
    var = jnp.mean((o - mean) * (o - mean), axis=0, keepdims=True)
    out_ref[...] = (o - mean) / jnp.sqrt(var + 1e-5) * gam_ref[...] + bet_ref[...]


def _heads_body(o_ref, dw1_ref, db1_ref, da1_ref, dw2_ref, db2_ref, da2_ref,
                sw1_ref, sb1_ref, sa1_ref, sw2_ref, sb2_ref,
                dos_ref, scal_ref):
    o = o_ref[...]
    h = jnp.dot(o, dw1_ref[...], preferred_element_type=F32) + db1_ref[...]
    h = jnp.where(h >= 0.0, h, da1_ref[0, 0] * h)
    d = jnp.dot(h, dw2_ref[...], preferred_element_type=F32) + db2_ref[...]
    dos_ref[...] = jnp.where(d >= 0.0, d, da2_ref[0, 0] * d)
    s = jnp.dot(o, sw1_ref[...], preferred_element_type=F32) + sb1_ref[...]
    s = jnp.where(s >= 0.0, s, sa1_ref[0, 0] * s)
    scal_ref[...] = jnp.dot(s, sw2_ref[...], preferred_element_type=F32) + sb2_ref[...]


_prep = pl.pallas_call(
    _prep_body,
    out_shape=[
        jax.ShapeDtypeStruct((N, D), F32),
        jax.ShapeDtypeStruct((N, D), F32),
        jax.ShapeDtypeStruct((N, D), F32),
    ],
)

_BE = 4000

_edge_feat = pl.pallas_call(
    _edge_feat_body,
    grid=(E // _BE,),
    in_specs=[
        pl.BlockSpec((_BE, DE), lambda i: (i, 0)),
        pl.BlockSpec((DE, DE), lambda i: (0, 0)),
        pl.BlockSpec((1, DE), lambda i: (0, 0)),
        pl.BlockSpec((1, 1), lambda i: (0, 0)),
        pl.BlockSpec((DE, D), lambda i: (0, 0)),
        pl.BlockSpec((1, D), lambda i: (0, 0)),
    ],
    out_specs=pl.BlockSpec((_BE, D), lambda i: (i, 0)),
    out_shape=jax.ShapeDtypeStruct((E, D), F32),
)

_cnt_inv = pl.pallas_call(
    _cnt_inv_body,
    out_shape=jax.ShapeDtypeStruct((N, 1), F32),
)

_node_ab = pl.pallas_call(
    _node_body_ab,
    out_shape=[
        jax.ShapeDtypeStruct((N, D), F32),
        jax.ShapeDtypeStruct((N, D), F32),
        jax.ShapeDtypeStruct((N, D), F32),
    ],
)

_node_last = pl.pallas_call(
    _node_body_last,
    out_shape=jax.ShapeDtypeStruct((N, D), F32),
)

_heads = pl.pallas_call(
    _heads_body,
    out_shape=[
        jax.ShapeDtypeStruct((N, OUT), F32),
        jax.ShapeDtypeStruct((N, 1), F32),
    ],
)


def kernel(x, edge_index, edge_vec, pre_W, pre_b, pre_a,
           gc_mlp2_W, gc_mlp2_b, gc_mlp2_a, gc_mlp_W, gc_mlp_b, gc_mlp_a,
           bn_gamma, bn_beta, dos_W1, dos_b1, dos_a1, dos_W2, dos_b2, dos_a2,
           sc_W1, sc_b1, sc_a1, sc_W2, sc_b2):
    src = edge_index[0]
    dst = edge_index[1]
    wd = gc_mlp_W[:, :D, :]
    ws = gc_mlp_W[:, D:2 * D, :]
    we = gc_mlp_W[:, 2 * D:, :]

    zero_nd = jnp.zeros((NP, D), F32)
    zero_cnt = jnp.zeros((NP, CW), F32)
    ones_rows = jnp.ones((K, CW), F32)

    out, a_rows, b_rows = _prep(
        x, pre_W, pre_b.reshape(1, D), pre_a.reshape(1, 1), wd[0], ws[0])
    inv = _cnt_inv(_sc_cnt(dst, zero_cnt, ones_rows))

    for i in range(GC):
        c_rows = _edge_feat(
            edge_vec, gc_mlp2_W[i], gc_mlp2_b[i].reshape(1, DE),
            gc_mlp2_a[i].reshape(1, 1), we[i], gc_mlp_b[i].reshape(1, D))
        alpha = jnp.broadcast_to(gc_mlp_a[i], (L,)).astype(F32)
        ssum2 = _sc_edge(a_rows, b_rows, c_rows, src, dst, alpha, zero_nd)
        gam = bn_gamma[i].reshape(1, D)
        bet = bn_beta[i].reshape(1, D)
        if i < GC - 1:
            out, a_rows, b_rows = _node_ab(
                ssum2, inv, out, gam, bet, wd[i + 1], ws[i + 1])
        else:
            out = _node_last(ssum2, inv, out, gam, bet)

    dos, scal = _heads(
        out, dos_W1, dos_b1.reshape(1, 64), dos_a1.reshape(1, 1),
        dos_W2, dos_b2.reshape(1, OUT), dos_a2.reshape(1, 1),
        sc_W1, sc_b1.reshape(1, 100), sc_a1.reshape(1, 1),
        sc_W2, sc_b2.reshape(1, 1))
    return dos, scal.reshape(N)


# speedup vs baseline: 4.7559x; 1.1631x over previous
import functools

import jax
import jax.numpy as jnp
from jax import lax
from jax.experimental import pallas as pl
from jax.experimental.pallas import tpu as pltpu
from jax.experimental.pallas import tpu_sc as plsc

N = 10000
E = 320000
D = 128
DE = 50
OUT = 200
GC = 3

NC = 2
NS = 16
NW = NC * NS
L = 16
CW = 128

EPW = E // NW
K = 40
KP = 48
PAD_ROW = 10224
NCHUNK = EPW // K
NP = 10240
RPS = NP // NS

F32 = jnp.float32


def _sc_edge_body(a_hbm, b_hbm, c_hbm, src_hbm, dst_hbm, alpha_hbm,
                  zero_nd_hbm,
                  ssum_out,
                  idx_s0, idx_s1, idx_d0, idx_d1, sidx0, sidx1,
                  arows0, arows1, brows0, brows1, crows0, crows1, alpha_v,
                  acc,
                  isem0, isem1, gsem0, gsem1, ssem0, ssem1):
    cid = lax.axis_index("c")
    sid = lax.axis_index("s")
    wid = sid * NC + cid

    r0 = sid * RPS
    pltpu.sync_copy(zero_nd_hbm.at[pl.ds(r0, RPS)], acc.at[pl.ds(r0, RPS)])
    pltpu.sync_copy(alpha_hbm, alpha_v)

    idx_s = (idx_s0, idx_s1)
    idx_d = (idx_d0, idx_d1)
    sidx = (sidx0, sidx1)
    arows = (arows0, arows1)
    brows = (brows0, brows1)
    crows = (crows0, crows1)
    isem = (isem0, isem1)
    gsem = (gsem0, gsem1)
    ssem = (ssem0, ssem1)

    padv = jnp.full((L,), PAD_ROW, jnp.int32)
    idx_d0[pl.ds(KP - L, L)] = padv
    idx_d1[pl.ds(KP - L, L)] = padv
    plsc.subcore_barrier()

    base = wid * EPW

    def fire_idx(c, p):
        off = base + c * K
        pltpu.async_copy(dst_hbm.at[pl.ds(off, K)],
                         idx_d[p].at[pl.ds(0, K)], isem[p])
        pltpu.async_copy(src_hbm.at[pl.ds(off, K)], idx_s[p], isem[p])

    def wait_idx(p):
        pltpu.make_async_copy(dst_hbm.at[pl.ds(0, K)],
                              idx_d[p].at[pl.ds(0, K)], isem[p]).wait()
        pltpu.make_async_copy(src_hbm.at[pl.ds(0, K)], idx_s[p],
                              isem[p]).wait()

    def fire_gathers(c, p):
        off = base + c * K
        pltpu.async_copy(a_hbm.at[idx_d[p].at[pl.ds(0, K)]],
                         arows[p].at[pl.ds(0, K)], gsem[p])
        pltpu.async_copy(b_hbm.at[idx_s[p]], brows[p], gsem[p])
        pltpu.async_copy(c_hbm.at[pl.ds(off, K)], crows[p], gsem[p])

    def wait_gathers(p):
        pltpu.make_async_copy(a_hbm.at[pl.ds(0, K)],
                              arows[p].at[pl.ds(0, K)], gsem[p]).wait()
        pltpu.make_async_copy(b_hbm.at[pl.ds(0, K)], brows[p],
                              gsem[p]).wait()
        pltpu.make_async_copy(c_hbm.at[pl.ds(0, K)], crows[p],
                              gsem[p]).wait()

    def fire_scatter(p):
        pltpu.async_copy(arows[p], acc.at[sidx[p]], ssem[p], add=True)

    def wait_scatter(p):
        pltpu.make_async_copy(zero_nd_hbm.at[pl.ds(0, KP)], arows[p],
                              ssem[p]).wait()

    def chunk_steps(c, p, skip_w6, fire_i, fire_g):
        q = 1 - p
        wait_gathers(p)
        for g in range(KP // L):
            s = pl.ds(g * L, L)
            sidx[p][s] = idx_d[p][s]
        if fire_i:
            fire_idx(c + 2, p)
        if fire_g:
            wait_idx(q)
            if not skip_w6:
                wait_scatter(q)
            fire_gathers(c + 1, q)
        al = alpha_v[...]

        def row(r, c2):
            for g in range(D // L):
                s = pl.ds(g * L, L)
                u = arows[p][r, s] + brows[p][r, s] + crows[p][r, s]
                arows[p][r, s] = jnp.where(u >= 0.0, u, al * u)
            return c2

        lax.fori_loop(0, K, row, 0)
        fire_scatter(p)

    fire_idx(0, 0)
    wait_idx(0)
    fire_idx(1, 1)
    fire_gathers(0, 0)
    chunk_steps(0, 0, True, True, True)
    chunk_steps(1, 1, False, True, True)

    def pair(t, carry):
        c0 = 2 * t
        chunk_steps(c0, 0, False, True, True)
        chunk_steps(c0 + 1, 1, False, True, True)
        return carry

    lax.fori_loop(1, NCHUNK // 2 - 1, pair, 0)
    chunk_steps(NCHUNK - 2, 0, False, False, True)
    chunk_steps(NCHUNK - 1, 1, False, False, False)
    wait_scatter(0)
    wait_scatter(1)

    plsc.subcore_barrier()
    pltpu.sync_copy(acc.at[pl.ds(r0, RPS)], ssum_out.at[cid, pl.ds(r0, RPS)])


_sc_edge = pl.kernel(
    _sc_edge_body,
    out_type=jax.ShapeDtypeStruct((NC, NP, D), F32),
    mesh=plsc.VectorSubcoreMesh(core_axis_name="c", subcore_axis_name="s"),
    scratch_types=[
        pltpu.VMEM((K,), jnp.int32),
        pltpu.VMEM((K,), jnp.int32),
        pltpu.VMEM((KP,), jnp.int32),
        pltpu.VMEM((KP,), jnp.int32),
        pltpu.VMEM((KP,), jnp.int32),
        pltpu.VMEM((KP,), jnp.int32),
        pltpu.VMEM((KP, D), F32),
        pltpu.VMEM((KP, D), F32),
        pltpu.VMEM((K, D), F32),
        pltpu.VMEM((K, D), F32),
        pltpu.VMEM((K, D), F32),
        pltpu.VMEM((K, D), F32),
        pltpu.VMEM((L,), F32),
        pltpu.VMEM_SHARED((NP, D), F32),
        pltpu.SemaphoreType.DMA,
        pltpu.SemaphoreType.DMA,
        pltpu.SemaphoreType.DMA,
        pltpu.SemaphoreType.DMA,
        pltpu.SemaphoreType.DMA,
        pltpu.SemaphoreType.DMA,
    ],
)


def _sc_cnt_body(dst_hbm, zero_cnt_hbm, ones_hbm, cnt_out,
                 idx_d, ones_v, cacc):
    cid = lax.axis_index("c")
    sid = lax.axis_index("s")
    wid = sid * NC + cid
    r0 = sid * RPS
    pltpu.sync_copy(zero_cnt_hbm.at[pl.ds(r0, RPS)], cacc.at[pl.ds(r0, RPS)])
    pltpu.sync_copy(ones_hbm, ones_v)
    plsc.subcore_barrier()
    base = wid * EPW

    def chunk(ci, carry):
        pltpu.sync_copy(dst_hbm.at[pl.ds(base + ci * K, K)], idx_d)
        pltpu.sync_copy(ones_v, cacc.at[idx_d], add=True)
        return carry

    lax.fori_loop(0, NCHUNK, chunk, 0)
    plsc.subcore_barrier()
    pltpu.sync_copy(cacc.at[pl.ds(r0, RPS)], cnt_out.at[cid, pl.ds(r0, RPS)])


_sc_cnt = pl.kernel(
    _sc_cnt_body,
    out_type=jax.ShapeDtypeStruct((NC, NP, CW), F32),
    mesh=plsc.VectorSubcoreMesh(core_axis_name="c", subcore_axis_name="s"),
    scratch_types=[
        pltpu.VMEM((K,), jnp.int32),
        pltpu.VMEM((K, CW), F32),
        pltpu.VMEM_SHARED((NP, CW), F32),
    ],
)


def _prep_body(x_ref, w_ref, b_ref, a_ref, wd_ref, ws_ref,
               out_ref, ao_ref, bo_ref):
    h = jnp.dot(x_ref[...], w_ref[...], preferred_element_type=F32) + b_ref[...]
    o = jnp.where(h >= 0.0, h, a_ref[0, 0] * h)
    out_ref[...] = o
    ao_ref[...] = jnp.dot(o, wd_ref[...], preferred_element_type=F32)
    bo_ref[...] = jnp.dot(o, ws_ref[...], preferred_element_type=F32)


def _edge_feat_body(ev_ref, w2_ref, b2_ref, a2_ref, we_ref, bm_ref, c_ref):
    e = jnp.dot(ev_ref[...], w2_ref[...], preferred_element_type=F32) + b2_ref[...]
    e = jnp.where(e >= 0.0, e, a2_ref[0, 0] * e)
    c_ref[...] = jnp.dot(e, we_ref[...], preferred_element_type=F32) + bm_ref[...]


def _cnt_inv_body(cnt_ref, inv_ref):
    c = cnt_ref[0, :N, 0:1] + cnt_ref[1, :N, 0:1]
    inv_ref[...] = 1.0 / jnp.maximum(c, 1.0)


def _node_body_ab(ssum_ref, inv_ref, prev_ref, gam_ref, bet_ref,
                  wd_ref, ws_ref, out_ref, ao_ref, bo_ref):
    s = ssum_ref[0, :N] + ssum_ref[1, :N]
    o = s * inv_ref[...] + prev_ref[...]
    mean = jnp.mean(o, axis=0, keepdims=True)
    var = jnp.mean((o - mean) * (o - mean), axis=0, keepdims=True)
    o = (o - mean) / jnp.sqrt(var + 1e-5) * gam_ref[...] + bet_ref[...]
    out_ref[...] = o
    ao_ref[...] = jnp.dot(o, wd_ref[...], preferred_element_type=F32)
    bo_ref[...] = jnp.dot(o, ws_ref[...], preferred_element_type=F32)


def _node_body_last(ssum_ref, inv_ref, prev_ref, gam_ref, bet_ref, out_ref):
    s = ssum_ref[0, :N] + ssum_ref[1, :N]
    o = s * inv_ref[...] + prev_ref[...]
    mean = jnp.mean(o, axis=0, keepdims=True)
    var = jnp.mean((o - mean) * (o - mean), axis=0, keepdims=True)
    out_ref[...] = (o - mean) / jnp.sqrt(var + 1e-5) * gam_ref[...] + bet_ref[...]


def _heads_body(o_ref, dw1_ref, db1_ref, da1_ref, dw2_ref, db2_ref, da2_ref,
                sw1_ref, sb1_ref, sa1_ref, sw2_ref, sb2_ref,
                dos_ref, scal_ref):
    o = o_ref[...]
    h = jnp.dot(o, dw1_ref[...], preferred_element_type=F32) + db1_ref[...]
    h = jnp.where(h >= 0.0, h, da1_ref[0, 0] * h)
    d = jnp.dot(h, dw2_ref[...], preferred_element_type=F32) + db2_ref[...]
    dos_ref[...] = jnp.where(d >= 0.0, d, da2_ref[0, 0] * d)
    s = jnp.dot(o, sw1_ref[...], preferred_element_type=F32) + sb1_ref[...]
    s = jnp.where(s >= 0.0, s, sa1_ref[0, 0] * s)
    scal_ref[...] = jnp.dot(s, sw2_ref[...], preferred_element_type=F32) + sb2_ref[...]


_prep = pl.pallas_call(
    _prep_body,
    out_shape=[
        jax.ShapeDtypeStruct((N, D), F32),
        jax.ShapeDtypeStruct((N, D), F32),
        jax.ShapeDtypeStruct((N, D), F32),
    ],
)

_BE = 4000

_edge_feat = pl.pallas_call(
    _edge_feat_body,
    grid=(E // _BE,),
    in_specs=[
        pl.BlockSpec((_BE, DE), lambda i: (i, 0)),
        pl.BlockSpec((DE, DE), lambda i: (0, 0)),
        pl.BlockSpec((1, DE), lambda i: (0, 0)),
        pl.BlockSpec((1, 1), lambda i: (0, 0)),
        pl.BlockSpec((DE, D), lambda i: (0, 0)),
        pl.BlockSpec((1, D), lambda i: (0, 0)),
    ],
    out_specs=pl.BlockSpec((_BE, D), lambda i: (i, 0)),
    out_shape=jax.ShapeDtypeStruct((E, D), F32),
)

_cnt_inv = pl.pallas_call(
    _cnt_inv_body,
    out_shape=jax.ShapeDtypeStruct((N, 1), F32),
)

_node_ab = pl.pallas_call(
    _node_body_ab,
    out_shape=[
        jax.ShapeDtypeStruct((N, D), F32),
        jax.ShapeDtypeStruct((N, D), F32),
        jax.ShapeDtypeStruct((N, D), F32),
    ],
)

_node_last = pl.pallas_call(
    _node_body_last,
    out_shape=jax.ShapeDtypeStruct((N, D), F32),
)

_heads = pl.pallas_call(
    _heads_body,
    out_shape=[
        jax.ShapeDtypeStruct((N, OUT), F32),
        jax.ShapeDtypeStruct((N, 1), F32),
    ],
)


def kernel(x, edge_index, edge_vec, pre_W, pre_b, pre_a,
           gc_mlp2_W, gc_mlp2_b, gc_mlp2_a, gc_mlp_W, gc_mlp_b, gc_mlp_a,
           bn_gamma, bn_beta, dos_W1, dos_b1, dos_a1, dos_W2, dos_b2, dos_a2,
           sc_W1, sc_b1, sc_a1, sc_W2, sc_b2):
    src = edge_index[0]
    dst = edge_index[1]
    wd = gc_mlp_W[:, :D, :]
    ws = gc_mlp_W[:, D:2 * D, :]
    we = gc_mlp_W[:, 2 * D:, :]

    zero_nd = jnp.zeros((NP, D), F32)
    zero_cnt = jnp.zeros((NP, CW), F32)
    ones_rows = jnp.ones((K, CW), F32)

    out, a_rows, b_rows = _prep(
        x, pre_W, pre_b.reshape(1, D), pre_a.reshape(1, 1), wd[0], ws[0])
    inv = _cnt_inv(_sc_cnt(dst, zero_cnt, ones_rows))

    for i in range(GC):
        c_rows = _edge_feat(
            edge_vec, gc_mlp2_W[i], gc_mlp2_b[i].reshape(1, DE),
            gc_mlp2_a[i].reshape(1, 1), we[i], gc_mlp_b[i].reshape(1, D))
        alpha = jnp.broadcast_to(gc_mlp_a[i], (L,)).astype(F32)
        ssum2 = _sc_edge(a_rows, b_rows, c_rows, src, dst, alpha, zero_nd)
        gam = bn_gamma[i].reshape(1, D)
        bet = bn_beta[i].reshape(1, D)
        if i < GC - 1:
            out, a_rows, b_rows = _node_ab(
                ssum2, inv, out, gam, bet, wd[i + 1], ws[i + 1])
        else:
            out = _node_last(ssum2, inv, out, gam, bet)

    dos, scal = _heads(
        out, dos_W1, dos_b1.reshape(1, 64), dos_a1.reshape(1, 1),
        dos_W2, dos_b2.reshape(1, OUT), dos_a2.reshape(1, 1),
        sc_W1, sc_b1.reshape(1, 100), sc_a1.reshape(1, 1),
        sc_W2, sc_b2.reshape(1, 1))
    return dos, scal.reshape(N)
